# baseline (speedup 1.0000x reference)
import functools

import jax
import jax.numpy as jnp
from jax import lax
from jax.experimental import pallas as pl
from jax.experimental.pallas import tpu as pltpu
from jax.experimental.pallas import tpu_sc as plsc

KP_EXTENT = 0.6


def _sc_gather(ftab, spx, spy, spz, idx_flat):
    E = idx_flat.shape[0]
    N, CW = ftab.shape
    assert ftab.dtype == jnp.float32
    info = plsc.get_sparse_core_info()
    NC, NS = info.num_cores, info.num_subcores
    NW = NC * NS
    per_w = E // NW
    for CH in (80, 48, 16):
        iters = per_w // CH
        if per_w % CH == 0 and iters % 4 == 1 and iters >= 9:
            break
    assert per_w % CH == 0 and E % NW == 0 and CH % 16 == 0
    assert iters % 4 == 1 and iters >= 9

    mesh = plsc.VectorSubcoreMesh(core_axis_name="c", subcore_axis_name="s")

    @functools.partial(
        pl.kernel,
        mesh=mesh,
        compiler_params=pltpu.CompilerParams(needs_layout_passes=False),
        out_type=[
            jax.ShapeDtypeStruct((E, CW), jnp.float32),
            jax.ShapeDtypeStruct((E,), jnp.float32),
            jax.ShapeDtypeStruct((E,), jnp.float32),
            jax.ShapeDtypeStruct((E,), jnp.float32),
        ],
        scratch_types=[
            pltpu.VMEM((per_w,), jnp.int32),
            pltpu.VMEM((4, CH, CW), jnp.float32),
            pltpu.VMEM((N,), jnp.float32),
            pltpu.VMEM((N,), jnp.float32),
            pltpu.VMEM((N,), jnp.float32),
            pltpu.VMEM((4, CH), jnp.float32),
            pltpu.VMEM((4, CH), jnp.float32),
            pltpu.VMEM((4, CH), jnp.float32),
            pltpu.SemaphoreType.DMA,
            pltpu.SemaphoreType.DMA,
            pltpu.SemaphoreType.DMA,
            pltpu.SemaphoreType.DMA,
            pltpu.SemaphoreType.DMA,
            pltpu.SemaphoreType.DMA,
        ],
    )
    def gather_kernel(ftab_hbm, spx_hbm, spy_hbm, spz_hbm, idx_hbm,
                      outf_hbm, outx_hbm, outy_hbm, outz_hbm,
                      idx_v, rows_v, spx_v, spy_v, spz_v, sx_v, sy_v, sz_v,
                      semg0, semg1, semw0, semw1, semw2, semw3):
        wid = lax.axis_index("s") * NC + lax.axis_index("c")
        base0 = wid * per_w
        semg = (semg0, semg1)
        semw = (semw0, semw1, semw2, semw3)
        pltpu.sync_copy(idx_hbm.at[pl.ds(base0, per_w)], idx_v)
        pltpu.sync_copy(spx_hbm, spx_v)
        pltpu.sync_copy(spy_hbm, spy_v)
        pltpu.sync_copy(spz_hbm, spz_v)

        def write_descs(i, buf):
            base = base0 + i * CH
            return (
                (sx_v.at[buf], outx_hbm.at[pl.ds(base, CH)]),
                (sy_v.at[buf], outy_hbm.at[pl.ds(base, CH)]),
                (sz_v.at[buf], outz_hbm.at[pl.ds(base, CH)]),
                (rows_v.at[buf], outf_hbm.at[pl.ds(base, CH)]),
            )

        def drain_writes(i, buf):
            for src, dst in write_descs(i, buf):
                pltpu.make_async_copy(src, dst, semw[buf]).wait()

        def start(i, buf, par, drain):
            if drain:
                drain_writes(i, buf)
            pltpu.async_copy(ftab_hbm.at[idx_v.at[pl.ds(i * CH, CH)]],
                             rows_v.at[buf], semg[par])

        def finish(i, buf, par):
            off = i * CH
            for t in range(CH // 16):
                iv = idx_v[pl.ds(off + t * 16, 16)]
                sx_v[buf, pl.ds(t * 16, 16)] = plsc.load_gather(spx_v, [iv])
                sy_v[buf, pl.ds(t * 16, 16)] = plsc.load_gather(spy_v, [iv])
                sz_v[buf, pl.ds(t * 16, 16)] = plsc.load_gather(spz_v, [iv])
            pltpu.make_async_copy(ftab_hbm.at[idx_v.at[pl.ds(off, CH)]],
                                  rows_v.at[buf], semg[par]).wait()
            for src, dst in write_descs(i, buf):
                pltpu.async_copy(src, dst, semw[buf])

        start(0, 0, 0, False)
        start(1, 1, 1, False)
        finish(0, 0, 0)
        start(2, 2, 0, False)
        finish(1, 1, 1)
        start(3, 3, 1, False)
        finish(2, 2, 0)
        start(4, 0, 0, True)
        finish(3, 3, 1)
        start(5, 1, 1, True)

        def body(t, carry):
            i = 4 * t
            finish(i, 0, 0)
            start(i + 2, 2, 0, True)
            finish(i + 1, 1, 1)
            start(i + 3, 3, 1, True)
            finish(i + 2, 2, 0)
            start(i + 4, 0, 0, True)
            finish(i + 3, 3, 1)
            start(i + 5, 1, 1, True)
            return carry

        lax.fori_loop(1, (iters - 5) // 4, body, 0)
        it = iters - 1
        finish(it - 4, 0, 0)
        start(it - 2, 2, 0, True)
        finish(it - 3, 1, 1)
        start(it - 1, 3, 1, True)
        finish(it - 2, 2, 0)
        start(it, 0, 0, True)
        finish(it - 1, 3, 1)
        finish(it, 0, 0)
        for j in (it - 3, it - 2, it - 1, it):
            drain_writes(j, j % 4)

    return gather_kernel(ftab, spx, spy, spz, idx_flat)


def _tc_body(fg_ref, sx_ref, sy_ref, sz_ref, qx_ref, qy_ref, qz_ref,
             kp_ref, wf_ref, out_ref, acc_ref, *, K, NN, B, P):
    EB = B * NN
    kp = kp_ref[...]
    kpx = kp[:, 0:1]
    kpy = kp[:, 1:2]
    kpz = kp[:, 2:3]
    sx = sx_ref[...].reshape(1, EB)
    sy = sy_ref[...].reshape(1, EB)
    sz = sz_ref[...].reshape(1, EB)
    qx = qx_ref[...].reshape(1, EB)
    qy = qy_ref[...].reshape(1, EB)
    qz = qz_ref[...].reshape(1, EB)
    dx = (sx - qx) - kpx
    dy = (sy - qy) - kpy
    dz = (sz - qz) - kpz
    d2 = dx * dx + dy * dy + dz * dz
    w = jnp.maximum(1.0 - jnp.sqrt(d2) * (1.0 / KP_EXTENT), 0.0)

    rows = K * P
    cols = P * NN
    r_iota = lax.broadcasted_iota(jnp.int32, (rows, cols), 0)
    c_iota = lax.broadcasted_iota(jnp.int32, (rows, cols), 1)
    mask = (r_iota % P) == (c_iota // NN)

    n_groups = B // P
    for g in range(n_groups):
        wg = w[:, g * cols:(g + 1) * cols]
        t8 = jnp.broadcast_to(wg[:, None, :], (K, P, cols))
        w2 = jnp.where(mask, t8.reshape(rows, cols), 0.0)
        fgg = fg_ref[pl.ds(g * cols, cols), :]
        accg = jnp.dot(w2, fgg, preferred_element_type=jnp.float32)
        for k in range(K):
            acc_ref[pl.ds(g * P, P), pl.ds(k * 128, 128)] = (
                accg[k * P:(k + 1) * P, :])
    out_ref[...] = jnp.dot(acc_ref[...], wf_ref[...],
                           preferred_element_type=jnp.float32)


def _tc_compute(featg, sxg, syg, szg, qxe, qye, qze, kp, wflat):
    E, C = featg.shape
    K = kp.shape[0]
    B = 200
    NN = 32
    EB = B * NN
    nblk = E // EB
    body = functools.partial(_tc_body, K=K, NN=NN, B=B, P=8)
    edge_spec = pl.BlockSpec((1, 1, EB), lambda i: (i, 0, 0))
    assert featg.dtype == jnp.float32
    return pl.pallas_call(
        body,
        grid=(nblk,),
        in_specs=[
            pl.BlockSpec((EB, C), lambda i: (i, 0)),
            edge_spec, edge_spec, edge_spec,
            edge_spec, edge_spec, edge_spec,
            pl.BlockSpec((K, 3), lambda i: (0, 0)),
            pl.BlockSpec((K * C, C), lambda i: (0, 0)),
        ],
        out_specs=pl.BlockSpec((B, C), lambda i: (i, 0)),
        out_shape=jax.ShapeDtypeStruct((nblk * B, C), jnp.float32),
        scratch_shapes=[pltpu.VMEM((B, K * C), jnp.float32)],
    )(featg, sxg, syg, szg, qxe, qye, qze, kp, wflat)


def kernel(query_points, support_points, neighbors_indices, features, wts,
           kernel_points):
    N, C = features.shape
    NN = neighbors_indices.shape[1]
    E = N * NN
    K = kernel_points.shape[0]
    idx_flat = neighbors_indices.reshape(-1).astype(jnp.int32)
    spx = support_points[:, 0]
    spy = support_points[:, 1]
    spz = support_points[:, 2]
    NCHUNK = 5
    PC = N // NCHUNK
    EC = PC * NN
    outs = []
    for c in range(NCHUNK):
        idx_c = lax.dynamic_slice_in_dim(idx_flat, c * EC, EC)
        featg, sxg, syg, szg = _sc_gather(features, spx, spy, spz, idx_c)
        qp_c = lax.dynamic_slice_in_dim(query_points, c * PC, PC)
        outs.append(_chunk_tc(featg, sxg, syg, szg, qp_c, kernel_points, wts,
                              NN, EC))
    return jnp.concatenate(outs, axis=0)


def _chunk_tc(featg, sxg, syg, szg, qp_c, kernel_points, wts, NN, EC):
    PC, C = qp_c.shape[0], featg.shape[1]
    K = kernel_points.shape[0]
    EB = 200 * NN
    nblk = EC // EB
    qxe = jnp.repeat(qp_c[:, 0], NN).reshape(nblk, 1, EB)
    qye = jnp.repeat(qp_c[:, 1], NN).reshape(nblk, 1, EB)
    qze = jnp.repeat(qp_c[:, 2], NN).reshape(nblk, 1, EB)
    wflat = wts.reshape(K * C, C)
    return _tc_compute(featg, sxg.reshape(nblk, 1, EB),
                       syg.reshape(nblk, 1, EB), szg.reshape(nblk, 1, EB),
                       qxe, qye, qze, kernel_points, wflat)


# speedup vs baseline: 1.2709x; 1.2709x over previous
import functools

import jax
import jax.numpy as jnp
from jax import lax
from jax.experimental import pallas as pl
from jax.experimental.pallas import tpu as pltpu
from jax.experimental.pallas import tpu_sc as plsc

KP_EXTENT = 0.6


def _sc_gather(ftab, spx, spy, spz, idx_flat):
    E = idx_flat.shape[0]
    N, CW = ftab.shape
    assert ftab.dtype == jnp.float32
    info = plsc.get_sparse_core_info()
    NC, NS = info.num_cores, info.num_subcores
    NW = NC * NS
    per_w = E // NW
    for CH in (80, 48, 16):
        iters = per_w // CH
        if per_w % CH == 0 and iters % 4 == 1 and iters >= 9:
            break
    assert per_w % CH == 0 and E % NW == 0 and CH % 16 == 0
    assert iters % 4 == 1 and iters >= 9

    mesh = plsc.VectorSubcoreMesh(core_axis_name="c", subcore_axis_name="s")

    @functools.partial(
        pl.kernel,
        mesh=mesh,
        compiler_params=pltpu.CompilerParams(needs_layout_passes=False),
        out_type=[
            jax.ShapeDtypeStruct((E, CW), jnp.float32),
            jax.ShapeDtypeStruct((E,), jnp.float32),
            jax.ShapeDtypeStruct((E,), jnp.float32),
            jax.ShapeDtypeStruct((E,), jnp.float32),
        ],
        scratch_types=[
            pltpu.VMEM((per_w,), jnp.int32),
            pltpu.VMEM((4, CH, CW), jnp.float32),
            pltpu.VMEM((N,), jnp.float32),
            pltpu.VMEM((N,), jnp.float32),
            pltpu.VMEM((N,), jnp.float32),
            pltpu.VMEM((4, CH), jnp.float32),
            pltpu.VMEM((4, CH), jnp.float32),
            pltpu.VMEM((4, CH), jnp.float32),
            pltpu.SemaphoreType.DMA,
            pltpu.SemaphoreType.DMA,
            pltpu.SemaphoreType.DMA,
            pltpu.SemaphoreType.DMA,
            pltpu.SemaphoreType.DMA,
            pltpu.SemaphoreType.DMA,
        ],
    )
    def gather_kernel(ftab_hbm, spx_hbm, spy_hbm, spz_hbm, idx_hbm,
                      outf_hbm, outx_hbm, outy_hbm, outz_hbm,
                      idx_v, rows_v, spx_v, spy_v, spz_v, sx_v, sy_v, sz_v,
                      semg0, semg1, semw0, semw1, semw2, semw3):
        wid = lax.axis_index("s") * NC + lax.axis_index("c")
        base0 = wid * per_w
        semg = (semg0, semg1)
        semw = (semw0, semw1, semw2, semw3)
        pltpu.sync_copy(idx_hbm.at[pl.ds(base0, per_w)], idx_v)
        pltpu.sync_copy(spx_hbm, spx_v)
        pltpu.sync_copy(spy_hbm, spy_v)
        pltpu.sync_copy(spz_hbm, spz_v)

        def write_descs(i, buf):
            base = base0 + i * CH
            return (
                (sx_v.at[buf], outx_hbm.at[pl.ds(base, CH)]),
                (sy_v.at[buf], outy_hbm.at[pl.ds(base, CH)]),
                (sz_v.at[buf], outz_hbm.at[pl.ds(base, CH)]),
                (rows_v.at[buf], outf_hbm.at[pl.ds(base, CH)]),
            )

        def drain_writes(i, buf):
            for src, dst in write_descs(i, buf):
                pltpu.make_async_copy(src, dst, semw[buf]).wait()

        def start(i, buf, par, drain):
            if drain:
                drain_writes(i, buf)
            pltpu.async_copy(ftab_hbm.at[idx_v.at[pl.ds(i * CH, CH)]],
                             rows_v.at[buf], semg[par])

        def finish(i, buf, par):
            off = i * CH
            for t in range(CH // 16):
                iv = idx_v[pl.ds(off + t * 16, 16)]
                sx_v[buf, pl.ds(t * 16, 16)] = plsc.load_gather(spx_v, [iv])
                sy_v[buf, pl.ds(t * 16, 16)] = plsc.load_gather(spy_v, [iv])
                sz_v[buf, pl.ds(t * 16, 16)] = plsc.load_gather(spz_v, [iv])
            pltpu.make_async_copy(ftab_hbm.at[idx_v.at[pl.ds(off, CH)]],
                                  rows_v.at[buf], semg[par]).wait()
            for src, dst in write_descs(i, buf):
                pltpu.async_copy(src, dst, semw[buf])

        start(0, 0, 0, False)
        start(1, 1, 1, False)
        finish(0, 0, 0)
        start(2, 2, 0, False)
        finish(1, 1, 1)
        start(3, 3, 1, False)
        finish(2, 2, 0)
        start(4, 0, 0, True)
        finish(3, 3, 1)
        start(5, 1, 1, True)

        def body(t, carry):
            i = 4 * t
            finish(i, 0, 0)
            start(i + 2, 2, 0, True)
            finish(i + 1, 1, 1)
            start(i + 3, 3, 1, True)
            finish(i + 2, 2, 0)
            start(i + 4, 0, 0, True)
            finish(i + 3, 3, 1)
            start(i + 5, 1, 1, True)
            return carry

        lax.fori_loop(1, (iters - 5) // 4, body, 0)
        it = iters - 1
        finish(it - 4, 0, 0)
        start(it - 2, 2, 0, True)
        finish(it - 3, 1, 1)
        start(it - 1, 3, 1, True)
        finish(it - 2, 2, 0)
        start(it, 0, 0, True)
        finish(it - 1, 3, 1)
        finish(it, 0, 0)
        for j in (it - 3, it - 2, it - 1, it):
            drain_writes(j, j % 4)

    return gather_kernel(ftab, spx, spy, spz, idx_flat)


def _tc_body(fg_ref, sx_ref, sy_ref, sz_ref, qx_ref, qy_ref, qz_ref,
             kp_ref, wf_ref, out_ref, acc_ref, *, K, NN, B, P):
    EB = B * NN
    kp = kp_ref[...]
    kpx = kp[:, 0:1]
    kpy = kp[:, 1:2]
    kpz = kp[:, 2:3]
    sx = sx_ref[...].reshape(1, EB)
    sy = sy_ref[...].reshape(1, EB)
    sz = sz_ref[...].reshape(1, EB)
    qx = qx_ref[...].reshape(1, EB)
    qy = qy_ref[...].reshape(1, EB)
    qz = qz_ref[...].reshape(1, EB)
    dx = (sx - qx) - kpx
    dy = (sy - qy) - kpy
    dz = (sz - qz) - kpz
    d2 = dx * dx + dy * dy + dz * dz
    w = jnp.maximum(1.0 - jnp.sqrt(d2) * (1.0 / KP_EXTENT), 0.0)

    rows = K * P
    cols = P * NN
    r_iota = lax.broadcasted_iota(jnp.int32, (rows, cols), 0)
    c_iota = lax.broadcasted_iota(jnp.int32, (rows, cols), 1)
    mask = (r_iota % P) == (c_iota // NN)

    n_groups = B // P
    for g in range(n_groups):
        wg = w[:, g * cols:(g + 1) * cols]
        t8 = jnp.broadcast_to(wg[:, None, :], (K, P, cols))
        w2 = jnp.where(mask, t8.reshape(rows, cols), 0.0)
        fgg = fg_ref[pl.ds(g * cols, cols), :]
        accg = jnp.dot(w2, fgg, preferred_element_type=jnp.float32)
        for k in range(K):
            acc_ref[pl.ds(g * P, P), pl.ds(k * 128, 128)] = (
                accg[k * P:(k + 1) * P, :])
    out_ref[...] = jnp.dot(acc_ref[...], wf_ref[...],
                           preferred_element_type=jnp.float32)


def _tc_compute(featg, sxg, syg, szg, qxe, qye, qze, kp, wflat):
    E, C = featg.shape
    K = kp.shape[0]
    B = 200
    NN = 32
    EB = B * NN
    nblk = E // EB
    body = functools.partial(_tc_body, K=K, NN=NN, B=B, P=8)
    edge_spec = pl.BlockSpec((1, 1, EB), lambda i: (i, 0, 0))
    assert featg.dtype == jnp.float32
    return pl.pallas_call(
        body,
        grid=(nblk,),
        in_specs=[
            pl.BlockSpec((EB, C), lambda i: (i, 0)),
            edge_spec, edge_spec, edge_spec,
            edge_spec, edge_spec, edge_spec,
            pl.BlockSpec((K, 3), lambda i: (0, 0)),
            pl.BlockSpec((K * C, C), lambda i: (0, 0)),
        ],
        out_specs=pl.BlockSpec((B, C), lambda i: (i, 0)),
        out_shape=jax.ShapeDtypeStruct((nblk * B, C), jnp.float32),
        scratch_shapes=[pltpu.VMEM((B, K * C), jnp.float32)],
    )(featg, sxg, syg, szg, qxe, qye, qze, kp, wflat)


def kernel(query_points, support_points, neighbors_indices, features, wts,
           kernel_points):
    N, C = features.shape
    NN = neighbors_indices.shape[1]
    E = N * NN
    K = kernel_points.shape[0]
    idx_flat = neighbors_indices.reshape(-1).astype(jnp.int32)
    spx = support_points[:, 0]
    spy = support_points[:, 1]
    spz = support_points[:, 2]
    featg, sxg, syg, szg = _sc_gather(features, spx, spy, spz, idx_flat)
    return _chunk_tc(featg, sxg, syg, szg, query_points, kernel_points, wts,
                     NN, E)


def _chunk_tc(featg, sxg, syg, szg, qp_c, kernel_points, wts, NN, EC):
    PC, C = qp_c.shape[0], featg.shape[1]
    K = kernel_points.shape[0]
    EB = 200 * NN
    nblk = EC // EB
    qxe = jnp.repeat(qp_c[:, 0], NN).reshape(nblk, 1, EB)
    qye = jnp.repeat(qp_c[:, 1], NN).reshape(nblk, 1, EB)
    qze = jnp.repeat(qp_c[:, 2], NN).reshape(nblk, 1, EB)
    wflat = wts.reshape(K * C, C)
    return _tc_compute(featg, sxg.reshape(nblk, 1, EB),
                       syg.reshape(nblk, 1, EB), szg.reshape(nblk, 1, EB),
                       qxe, qye, qze, kernel_points, wflat)


# speedup vs baseline: 1.2888x; 1.0140x over previous
import functools

import jax
import jax.numpy as jnp
from jax import lax
from jax.experimental import pallas as pl
from jax.experimental.pallas import tpu as pltpu
from jax.experimental.pallas import tpu_sc as plsc

KP_EXTENT = 0.6


def _sc_gather(ftab, spx, spy, spz, idx_flat):
    E = idx_flat.shape[0]
    N, CW = ftab.shape
    assert ftab.dtype == jnp.float32
    info = plsc.get_sparse_core_info()
    NC, NS = info.num_cores, info.num_subcores
    NW = NC * NS
    per_w = E // NW
    for CH in (80, 48, 16):
        iters = per_w // CH
        if per_w % CH == 0 and iters % 4 == 1 and iters >= 9:
            break
    assert per_w % CH == 0 and E % NW == 0 and CH % 16 == 0
    assert iters % 4 == 1 and iters >= 9

    mesh = plsc.VectorSubcoreMesh(core_axis_name="c", subcore_axis_name="s")

    @functools.partial(
        pl.kernel,
        mesh=mesh,
        compiler_params=pltpu.CompilerParams(needs_layout_passes=False),
        out_type=[
            jax.ShapeDtypeStruct((E, CW), jnp.float32),
            jax.ShapeDtypeStruct((E,), jnp.float32),
            jax.ShapeDtypeStruct((E,), jnp.float32),
            jax.ShapeDtypeStruct((E,), jnp.float32),
        ],
        scratch_types=[
            pltpu.VMEM((per_w,), jnp.int32),
            pltpu.VMEM((4, CH, CW), jnp.float32),
            pltpu.VMEM((N,), jnp.float32),
            pltpu.VMEM((N,), jnp.float32),
            pltpu.VMEM((N,), jnp.float32),
            pltpu.VMEM((4, CH), jnp.float32),
            pltpu.VMEM((4, CH), jnp.float32),
            pltpu.VMEM((4, CH), jnp.float32),
            pltpu.SemaphoreType.DMA,
            pltpu.SemaphoreType.DMA,
            pltpu.SemaphoreType.DMA,
            pltpu.SemaphoreType.DMA,
            pltpu.SemaphoreType.DMA,
            pltpu.SemaphoreType.DMA,
        ],
    )
    def gather_kernel(ftab_hbm, spx_hbm, spy_hbm, spz_hbm, idx_hbm,
                      outf_hbm, outx_hbm, outy_hbm, outz_hbm,
                      idx_v, rows_v, spx_v, spy_v, spz_v, sx_v, sy_v, sz_v,
                      semg0, semg1, semw0, semw1, semw2, semw3):
        wid = lax.axis_index("s") * NC + lax.axis_index("c")
        base0 = wid * per_w
        semg = (semg0, semg1)
        semw = (semw0, semw1, semw2, semw3)
        pltpu.sync_copy(idx_hbm.at[pl.ds(base0, per_w)], idx_v)
        pltpu.sync_copy(spx_hbm, spx_v)
        pltpu.sync_copy(spy_hbm, spy_v)
        pltpu.sync_copy(spz_hbm, spz_v)

        def write_descs(i, buf):
            base = base0 + i * CH
            return (
                (sx_v.at[buf], outx_hbm.at[pl.ds(base, CH)]),
                (sy_v.at[buf], outy_hbm.at[pl.ds(base, CH)]),
                (sz_v.at[buf], outz_hbm.at[pl.ds(base, CH)]),
                (rows_v.at[buf], outf_hbm.at[pl.ds(base, CH)]),
            )

        def drain_writes(i, buf):
            for src, dst in write_descs(i, buf):
                pltpu.make_async_copy(src, dst, semw[buf]).wait()

        def start(i, buf, par, drain):
            if drain:
                drain_writes(i, buf)
            pltpu.async_copy(ftab_hbm.at[idx_v.at[pl.ds(i * CH, CH)]],
                             rows_v.at[buf], semg[par])

        def finish(i, buf, par):
            off = i * CH
            for t in range(CH // 16):
                iv = idx_v[pl.ds(off + t * 16, 16)]
                sx_v[buf, pl.ds(t * 16, 16)] = plsc.load_gather(spx_v, [iv])
                sy_v[buf, pl.ds(t * 16, 16)] = plsc.load_gather(spy_v, [iv])
                sz_v[buf, pl.ds(t * 16, 16)] = plsc.load_gather(spz_v, [iv])
            pltpu.make_async_copy(ftab_hbm.at[idx_v.at[pl.ds(off, CH)]],
                                  rows_v.at[buf], semg[par]).wait()
            for src, dst in write_descs(i, buf):
                pltpu.async_copy(src, dst, semw[buf])

        start(0, 0, 0, False)
        start(1, 1, 1, False)
        finish(0, 0, 0)
        start(2, 2, 0, False)
        finish(1, 1, 1)
        start(3, 3, 1, False)
        finish(2, 2, 0)
        start(4, 0, 0, True)
        finish(3, 3, 1)
        start(5, 1, 1, True)

        def body(t, carry):
            i = 4 * t
            finish(i, 0, 0)
            start(i + 2, 2, 0, True)
            finish(i + 1, 1, 1)
            start(i + 3, 3, 1, True)
            finish(i + 2, 2, 0)
            start(i + 4, 0, 0, True)
            finish(i + 3, 3, 1)
            start(i + 5, 1, 1, True)
            return carry

        lax.fori_loop(1, (iters - 5) // 4, body, 0)
        it = iters - 1
        finish(it - 4, 0, 0)
        start(it - 2, 2, 0, True)
        finish(it - 3, 1, 1)
        start(it - 1, 3, 1, True)
        finish(it - 2, 2, 0)
        start(it, 0, 0, True)
        finish(it - 1, 3, 1)
        finish(it, 0, 0)
        for j in (it - 3, it - 2, it - 1, it):
            drain_writes(j, j % 4)

    return gather_kernel(ftab, spx, spy, spz, idx_flat)


def _tc_body(fg_ref, sx_ref, sy_ref, sz_ref, qx_ref, qy_ref, qz_ref,
             kp_ref, wf_ref, out_ref, acc_ref, *, K, NN, B, P):
    EB = B * NN
    kp = kp_ref[...]
    kpx = kp[:, 0:1]
    kpy = kp[:, 1:2]
    kpz = kp[:, 2:3]
    sx = sx_ref[...].reshape(1, EB)
    sy = sy_ref[...].reshape(1, EB)
    sz = sz_ref[...].reshape(1, EB)
    qx = qx_ref[...].reshape(1, EB)
    qy = qy_ref[...].reshape(1, EB)
    qz = qz_ref[...].reshape(1, EB)
    dx = (sx - qx) - kpx
    dy = (sy - qy) - kpy
    dz = (sz - qz) - kpz
    d2 = dx * dx + dy * dy + dz * dz
    w = jnp.maximum(1.0 - jnp.sqrt(d2) * (1.0 / KP_EXTENT), 0.0)

    rows = K * P
    cols = P * NN
    r_iota = lax.broadcasted_iota(jnp.int32, (rows, cols), 0)
    c_iota = lax.broadcasted_iota(jnp.int32, (rows, cols), 1)
    mask = (r_iota % P) == (c_iota // NN)

    n_groups = B // P
    for g in range(n_groups):
        wg = w[:, g * cols:(g + 1) * cols]
        t8 = jnp.broadcast_to(wg[:, None, :], (K, P, cols))
        w2 = jnp.where(mask, t8.reshape(rows, cols), 0.0)
        fgg = fg_ref[pl.ds(g * cols, cols), :]
        accg = jnp.dot(w2, fgg, preferred_element_type=jnp.float32)
        for k in range(K):
            acc_ref[pl.ds(g * P, P), pl.ds(k * 128, 128)] = (
                accg[k * P:(k + 1) * P, :])
    out_ref[...] = jnp.dot(acc_ref[...], wf_ref[...],
                           preferred_element_type=jnp.float32)


def _tc_compute(featg, sxg, syg, szg, qxe, qye, qze, kp, wflat):
    E, C = featg.shape
    K = kp.shape[0]
    B = 200
    NN = 32
    EB = B * NN
    nblk = E // EB
    body = functools.partial(_tc_body, K=K, NN=NN, B=B, P=4)
    edge_spec = pl.BlockSpec((1, 1, EB), lambda i: (i, 0, 0))
    assert featg.dtype == jnp.float32
    return pl.pallas_call(
        body,
        grid=(nblk,),
        in_specs=[
            pl.BlockSpec((EB, C), lambda i: (i, 0)),
            edge_spec, edge_spec, edge_spec,
            edge_spec, edge_spec, edge_spec,
            pl.BlockSpec((K, 3), lambda i: (0, 0)),
            pl.BlockSpec((K * C, C), lambda i: (0, 0)),
        ],
        out_specs=pl.BlockSpec((B, C), lambda i: (i, 0)),
        out_shape=jax.ShapeDtypeStruct((nblk * B, C), jnp.float32),
        scratch_shapes=[pltpu.VMEM((B, K * C), jnp.float32)],
    )(featg, sxg, syg, szg, qxe, qye, qze, kp, wflat)


def kernel(query_points, support_points, neighbors_indices, features, wts,
           kernel_points):
    N, C = features.shape
    NN = neighbors_indices.shape[1]
    E = N * NN
    K = kernel_points.shape[0]
    idx_flat = neighbors_indices.reshape(-1).astype(jnp.int32)
    spx = support_points[:, 0]
    spy = support_points[:, 1]
    spz = support_points[:, 2]
    featg, sxg, syg, szg = _sc_gather(features, spx, spy, spz, idx_flat)
    return _chunk_tc(featg, sxg, syg, szg, query_points, kernel_points, wts,
                     NN, E)


def _chunk_tc(featg, sxg, syg, szg, qp_c, kernel_points, wts, NN, EC):
    PC, C = qp_c.shape[0], featg.shape[1]
    K = kernel_points.shape[0]
    EB = 200 * NN
    nblk = EC // EB
    qxe = jnp.repeat(qp_c[:, 0], NN).reshape(nblk, 1, EB)
    qye = jnp.repeat(qp_c[:, 1], NN).reshape(nblk, 1, EB)
    qze = jnp.repeat(qp_c[:, 2], NN).reshape(nblk, 1, EB)
    wflat = wts.reshape(K * C, C)
    return _tc_compute(featg, sxg.reshape(nblk, 1, EB),
                       syg.reshape(nblk, 1, EB), szg.reshape(nblk, 1, EB),
                       qxe, qye, qze, kernel_points, wflat)


# speedup vs baseline: 1.3782x; 1.0694x over previous
import functools

import jax
import jax.numpy as jnp
from jax import lax
from jax.experimental import pallas as pl
from jax.experimental.pallas import tpu as pltpu
from jax.experimental.pallas import tpu_sc as plsc

KP_EXTENT = 0.6


def _sc_gather(ftab, spx, spy, spz, idx_flat):
    E = idx_flat.shape[0]
    N, CW = ftab.shape
    assert ftab.dtype == jnp.float32
    info = plsc.get_sparse_core_info()
    NC, NS = info.num_cores, info.num_subcores
    NW = NC * NS
    per_w = E // NW
    for CH in (80, 48, 16):
        iters = per_w // CH
        if per_w % CH == 0 and iters % 4 == 1 and iters >= 9:
            break
    assert per_w % CH == 0 and E % NW == 0 and CH % 16 == 0
    assert iters % 4 == 1 and iters >= 9

    mesh = plsc.VectorSubcoreMesh(core_axis_name="c", subcore_axis_name="s")

    @functools.partial(
        pl.kernel,
        mesh=mesh,
        compiler_params=pltpu.CompilerParams(needs_layout_passes=False),
        out_type=[
            jax.ShapeDtypeStruct((E, CW), jnp.float32),
            jax.ShapeDtypeStruct((E,), jnp.float32),
            jax.ShapeDtypeStruct((E,), jnp.float32),
            jax.ShapeDtypeStruct((E,), jnp.float32),
        ],
        scratch_types=[
            pltpu.VMEM((per_w,), jnp.int32),
            pltpu.VMEM((4, CH, CW), jnp.float32),
            pltpu.VMEM((N,), jnp.float32),
            pltpu.VMEM((N,), jnp.float32),
            pltpu.VMEM((N,), jnp.float32),
            pltpu.VMEM((4, CH), jnp.float32),
            pltpu.VMEM((4, CH), jnp.float32),
            pltpu.VMEM((4, CH), jnp.float32),
            pltpu.SemaphoreType.DMA,
            pltpu.SemaphoreType.DMA,
            pltpu.SemaphoreType.DMA,
            pltpu.SemaphoreType.DMA,
            pltpu.SemaphoreType.DMA,
            pltpu.SemaphoreType.DMA,
        ],
    )
    def gather_kernel(ftab_hbm, spx_hbm, spy_hbm, spz_hbm, idx_hbm,
                      outf_hbm, outx_hbm, outy_hbm, outz_hbm,
                      idx_v, rows_v, spx_v, spy_v, spz_v, sx_v, sy_v, sz_v,
                      semg0, semg1, semw0, semw1, semw2, semw3):
        wid = lax.axis_index("s") * NC + lax.axis_index("c")
        base0 = wid * per_w
        semg = (semg0, semg1)
        semw = (semw0, semw1, semw2, semw3)
        pltpu.sync_copy(idx_hbm.at[pl.ds(base0, per_w)], idx_v)
        pltpu.sync_copy(spx_hbm, spx_v)
        pltpu.sync_copy(spy_hbm, spy_v)
        pltpu.sync_copy(spz_hbm, spz_v)

        def write_descs(i, buf):
            base = base0 + i * CH
            return (
                (sx_v.at[buf], outx_hbm.at[pl.ds(base, CH)]),
                (sy_v.at[buf], outy_hbm.at[pl.ds(base, CH)]),
                (sz_v.at[buf], outz_hbm.at[pl.ds(base, CH)]),
                (rows_v.at[buf], outf_hbm.at[pl.ds(base, CH)]),
            )

        def drain_writes(i, buf):
            for src, dst in write_descs(i, buf):
                pltpu.make_async_copy(src, dst, semw[buf]).wait()

        def start(i, buf, par, drain):
            if drain:
                drain_writes(i, buf)
            pltpu.async_copy(ftab_hbm.at[idx_v.at[pl.ds(i * CH, CH)]],
                             rows_v.at[buf], semg[par])

        def finish(i, buf, par):
            off = i * CH
            for t in range(CH // 16):
                iv = idx_v[pl.ds(off + t * 16, 16)]
                sx_v[buf, pl.ds(t * 16, 16)] = plsc.load_gather(spx_v, [iv])
                sy_v[buf, pl.ds(t * 16, 16)] = plsc.load_gather(spy_v, [iv])
                sz_v[buf, pl.ds(t * 16, 16)] = plsc.load_gather(spz_v, [iv])
            pltpu.make_async_copy(ftab_hbm.at[idx_v.at[pl.ds(off, CH)]],
                                  rows_v.at[buf], semg[par]).wait()
            for src, dst in write_descs(i, buf):
                pltpu.async_copy(src, dst, semw[buf])

        start(0, 0, 0, False)
        start(1, 1, 1, False)
        finish(0, 0, 0)
        start(2, 2, 0, False)
        finish(1, 1, 1)
        start(3, 3, 1, False)
        finish(2, 2, 0)
        start(4, 0, 0, True)
        finish(3, 3, 1)
        start(5, 1, 1, True)

        def body(t, carry):
            i = 4 * t
            finish(i, 0, 0)
            start(i + 2, 2, 0, True)
            finish(i + 1, 1, 1)
            start(i + 3, 3, 1, True)
            finish(i + 2, 2, 0)
            start(i + 4, 0, 0, True)
            finish(i + 3, 3, 1)
            start(i + 5, 1, 1, True)
            return carry

        lax.fori_loop(1, (iters - 5) // 4, body, 0)
        it = iters - 1
        finish(it - 4, 0, 0)
        start(it - 2, 2, 0, True)
        finish(it - 3, 1, 1)
        start(it - 1, 3, 1, True)
        finish(it - 2, 2, 0)
        start(it, 0, 0, True)
        finish(it - 1, 3, 1)
        finish(it, 0, 0)
        for j in (it - 3, it - 2, it - 1, it):
            drain_writes(j, j % 4)

    return gather_kernel(ftab, spx, spy, spz, idx_flat)


def _tc_body(fg_ref, sx_ref, sy_ref, sz_ref, qx_ref, qy_ref, qz_ref,
             kp_ref, wf_ref, out_ref, acc_ref, *, K, NN, B, P):
    EB = B * NN
    kp = kp_ref[...]
    kpx = kp[:, 0:1]
    kpy = kp[:, 1:2]
    kpz = kp[:, 2:3]
    sx = sx_ref[...].reshape(1, EB)
    sy = sy_ref[...].reshape(1, EB)
    sz = sz_ref[...].reshape(1, EB)
    qx = qx_ref[...].reshape(1, EB)
    qy = qy_ref[...].reshape(1, EB)
    qz = qz_ref[...].reshape(1, EB)
    dx = (sx - qx) - kpx
    dy = (sy - qy) - kpy
    dz = (sz - qz) - kpz
    d2 = dx * dx + dy * dy + dz * dz
    w = jnp.maximum(1.0 - jnp.sqrt(d2) * (1.0 / KP_EXTENT), 0.0)

    rows = K * P
    cols = P * NN
    r_iota = lax.broadcasted_iota(jnp.int32, (rows, cols), 0)
    c_iota = lax.broadcasted_iota(jnp.int32, (rows, cols), 1)
    mask = (r_iota % P) == (c_iota // NN)

    n_groups = B // P
    for g in range(n_groups):
        wg = w[:, g * cols:(g + 1) * cols]
        t8 = jnp.broadcast_to(wg[:, None, :], (K, P, cols))
        w2 = jnp.where(mask, t8.reshape(rows, cols), 0.0)
        fgg = fg_ref[pl.ds(g * cols, cols), :]
        accg = jnp.dot(w2, fgg, preferred_element_type=jnp.float32)
        for k in range(K):
            acc_ref[pl.ds(g * P, P), pl.ds(k * 128, 128)] = (
                accg[k * P:(k + 1) * P, :])
    out_ref[...] = jnp.dot(acc_ref[...], wf_ref[...],
                           preferred_element_type=jnp.float32)


def _tc_compute(featg, sxg, syg, szg, qxe, qye, qze, kp, wflat):
    E, C = featg.shape
    K = kp.shape[0]
    B = 400
    NN = 32
    EB = B * NN
    nblk = E // EB
    body = functools.partial(_tc_body, K=K, NN=NN, B=B, P=4)
    edge_spec = pl.BlockSpec((1, 1, EB), lambda i: (i, 0, 0))
    assert featg.dtype == jnp.float32
    return pl.pallas_call(
        body,
        grid=(nblk,),
        in_specs=[
            pl.BlockSpec((EB, C), lambda i: (i, 0)),
            edge_spec, edge_spec, edge_spec,
            edge_spec, edge_spec, edge_spec,
            pl.BlockSpec((K, 3), lambda i: (0, 0)),
            pl.BlockSpec((K * C, C), lambda i: (0, 0)),
        ],
        out_specs=pl.BlockSpec((B, C), lambda i: (i, 0)),
        out_shape=jax.ShapeDtypeStruct((nblk * B, C), jnp.float32),
        scratch_shapes=[pltpu.VMEM((B, K * C), jnp.float32)],
    )(featg, sxg, syg, szg, qxe, qye, qze, kp, wflat)


def kernel(query_points, support_points, neighbors_indices, features, wts,
           kernel_points):
    N, C = features.shape
    NN = neighbors_indices.shape[1]
    E = N * NN
    K = kernel_points.shape[0]
    idx_flat = neighbors_indices.reshape(-1).astype(jnp.int32)
    spx = support_points[:, 0]
    spy = support_points[:, 1]
    spz = support_points[:, 2]
    featg, sxg, syg, szg = _sc_gather(features, spx, spy, spz, idx_flat)
    return _chunk_tc(featg, sxg, syg, szg, query_points, kernel_points, wts,
                     NN, E)


def _chunk_tc(featg, sxg, syg, szg, qp_c, kernel_points, wts, NN, EC):
    PC, C = qp_c.shape[0], featg.shape[1]
    K = kernel_points.shape[0]
    EB = 400 * NN
    nblk = EC // EB
    qxe = jnp.repeat(qp_c[:, 0], NN).reshape(nblk, 1, EB)
    qye = jnp.repeat(qp_c[:, 1], NN).reshape(nblk, 1, EB)
    qze = jnp.repeat(qp_c[:, 2], NN).reshape(nblk, 1, EB)
    wflat = wts.reshape(K * C, C)
    return _tc_compute(featg, sxg.reshape(nblk, 1, EB),
                       syg.reshape(nblk, 1, EB), szg.reshape(nblk, 1, EB),
                       qxe, qye, qze, kernel_points, wflat)


# speedup vs baseline: 1.4292x; 1.0370x over previous
import functools

import jax
import jax.numpy as jnp
from jax import lax
from jax.experimental import pallas as pl
from jax.experimental.pallas import tpu as pltpu
from jax.experimental.pallas import tpu_sc as plsc

KP_EXTENT = 0.6


def _sc_gather(ftab, spx, spy, spz, idx_flat):
    E = idx_flat.shape[0]
    N, CW = ftab.shape
    assert ftab.dtype == jnp.float32
    info = plsc.get_sparse_core_info()
    NC, NS = info.num_cores, info.num_subcores
    NW = NC * NS
    per_w = E // NW
    for CH in (80, 48, 16):
        iters = per_w // CH
        if per_w % CH == 0 and iters % 4 == 1 and iters >= 9:
            break
    assert per_w % CH == 0 and E % NW == 0 and CH % 16 == 0
    assert iters % 4 == 1 and iters >= 9

    mesh = plsc.VectorSubcoreMesh(core_axis_name="c", subcore_axis_name="s")

    @functools.partial(
        pl.kernel,
        mesh=mesh,
        compiler_params=pltpu.CompilerParams(needs_layout_passes=False),
        out_type=[
            jax.ShapeDtypeStruct((E, CW), jnp.float32),
            jax.ShapeDtypeStruct((E,), jnp.float32),
            jax.ShapeDtypeStruct((E,), jnp.float32),
            jax.ShapeDtypeStruct((E,), jnp.float32),
        ],
        scratch_types=[
            pltpu.VMEM((per_w,), jnp.int32),
            pltpu.VMEM((4, CH, CW), jnp.float32),
            pltpu.VMEM((N,), jnp.float32),
            pltpu.VMEM((N,), jnp.float32),
            pltpu.VMEM((N,), jnp.float32),
            pltpu.VMEM((4, CH), jnp.float32),
            pltpu.VMEM((4, CH), jnp.float32),
            pltpu.VMEM((4, CH), jnp.float32),
            pltpu.SemaphoreType.DMA,
            pltpu.SemaphoreType.DMA,
            pltpu.SemaphoreType.DMA,
            pltpu.SemaphoreType.DMA,
            pltpu.SemaphoreType.DMA,
            pltpu.SemaphoreType.DMA,
        ],
    )
    def gather_kernel(ftab_hbm, spx_hbm, spy_hbm, spz_hbm, idx_hbm,
                      outf_hbm, outx_hbm, outy_hbm, outz_hbm,
                      idx_v, rows_v, spx_v, spy_v, spz_v, sx_v, sy_v, sz_v,
                      semg0, semg1, semw0, semw1, semw2, semw3):
        wid = lax.axis_index("s") * NC + lax.axis_index("c")
        base0 = wid * per_w
        semg = (semg0, semg1)
        semw = (semw0, semw1, semw2, semw3)
        pltpu.sync_copy(idx_hbm.at[pl.ds(base0, per_w)], idx_v)
        pltpu.sync_copy(spx_hbm, spx_v)
        pltpu.sync_copy(spy_hbm, spy_v)
        pltpu.sync_copy(spz_hbm, spz_v)

        def write_descs(i, buf):
            base = base0 + i * CH
            return (
                (sx_v.at[buf], outx_hbm.at[pl.ds(base, CH)]),
                (sy_v.at[buf], outy_hbm.at[pl.ds(base, CH)]),
                (sz_v.at[buf], outz_hbm.at[pl.ds(base, CH)]),
                (rows_v.at[buf], outf_hbm.at[pl.ds(base, CH)]),
            )

        def drain_writes(i, buf):
            for src, dst in write_descs(i, buf):
                pltpu.make_async_copy(src, dst, semw[buf]).wait()

        def start(i, buf, par, drain):
            if drain:
                drain_writes(i, buf)
            pltpu.async_copy(ftab_hbm.at[idx_v.at[pl.ds(i * CH, CH)]],
                             rows_v.at[buf], semg[par])

        def finish(i, buf, par):
            off = i * CH
            for t in range(CH // 16):
                iv = idx_v[pl.ds(off + t * 16, 16)]
                sx_v[buf, pl.ds(t * 16, 16)] = plsc.load_gather(spx_v, [iv])
                sy_v[buf, pl.ds(t * 16, 16)] = plsc.load_gather(spy_v, [iv])
                sz_v[buf, pl.ds(t * 16, 16)] = plsc.load_gather(spz_v, [iv])
            pltpu.make_async_copy(ftab_hbm.at[idx_v.at[pl.ds(off, CH)]],
                                  rows_v.at[buf], semg[par]).wait()
            for src, dst in write_descs(i, buf):
                pltpu.async_copy(src, dst, semw[buf])

        start(0, 0, 0, False)
        start(1, 1, 1, False)
        finish(0, 0, 0)
        start(2, 2, 0, False)
        finish(1, 1, 1)
        start(3, 3, 1, False)
        finish(2, 2, 0)
        start(4, 0, 0, True)
        finish(3, 3, 1)
        start(5, 1, 1, True)

        def body(t, carry):
            i = 4 * t
            finish(i, 0, 0)
            start(i + 2, 2, 0, True)
            finish(i + 1, 1, 1)
            start(i + 3, 3, 1, True)
            finish(i + 2, 2, 0)
            start(i + 4, 0, 0, True)
            finish(i + 3, 3, 1)
            start(i + 5, 1, 1, True)
            return carry

        lax.fori_loop(1, (iters - 5) // 4, body, 0)
        it = iters - 1
        finish(it - 4, 0, 0)
        start(it - 2, 2, 0, True)
        finish(it - 3, 1, 1)
        start(it - 1, 3, 1, True)
        finish(it - 2, 2, 0)
        start(it, 0, 0, True)
        finish(it - 1, 3, 1)
        finish(it, 0, 0)
        for j in (it - 3, it - 2, it - 1, it):
            drain_writes(j, j % 4)

    return gather_kernel(ftab, spx, spy, spz, idx_flat)


def _tc_body(fg_ref, sx_ref, sy_ref, sz_ref, qx_ref, qy_ref, qz_ref,
             kp_ref, wf_ref, out_ref, acc_ref, *, K, NN, B, P):
    EB = B * NN
    kp = kp_ref[...]
    kpx = kp[:, 0:1]
    kpy = kp[:, 1:2]
    kpz = kp[:, 2:3]
    sx = sx_ref[...].reshape(1, EB)
    sy = sy_ref[...].reshape(1, EB)
    sz = sz_ref[...].reshape(1, EB)
    qx = qx_ref[...].reshape(1, EB)
    qy = qy_ref[...].reshape(1, EB)
    qz = qz_ref[...].reshape(1, EB)
    dx = (sx - qx) - kpx
    dy = (sy - qy) - kpy
    dz = (sz - qz) - kpz
    d2 = dx * dx + dy * dy + dz * dz
    w = jnp.maximum(1.0 - jnp.sqrt(d2) * (1.0 / KP_EXTENT), 0.0)

    rows = K * P
    cols = P * NN
    r_iota = lax.broadcasted_iota(jnp.int32, (rows, cols), 0)
    c_iota = lax.broadcasted_iota(jnp.int32, (rows, cols), 1)
    mask = (r_iota % P) == (c_iota // NN)

    n_groups = B // P
    for g in range(n_groups):
        wg = w[:, g * cols:(g + 1) * cols]
        t8 = jnp.broadcast_to(wg[:, None, :], (K, P, cols))
        w2 = jnp.where(mask, t8.reshape(rows, cols), 0.0)
        fgg = fg_ref[pl.ds(g * cols, cols), :]
        accg = jnp.dot(w2, fgg, preferred_element_type=jnp.float32)
        for k in range(K):
            acc_ref[pl.ds(g * P, P), pl.ds(k * 128, 128)] = (
                accg[k * P:(k + 1) * P, :])
    out_ref[...] = jnp.dot(acc_ref[...], wf_ref[...],
                           preferred_element_type=jnp.float32)


def _tc_compute(featg, sxg, syg, szg, qxe, qye, qze, kp, wflat):
    E, C = featg.shape
    K = kp.shape[0]
    B = 1000
    NN = 32
    EB = B * NN
    nblk = E // EB
    body = functools.partial(_tc_body, K=K, NN=NN, B=B, P=4)
    edge_spec = pl.BlockSpec((1, 1, EB), lambda i: (i, 0, 0))
    assert featg.dtype == jnp.float32
    return pl.pallas_call(
        body,
        grid=(nblk,),
        in_specs=[
            pl.BlockSpec((EB, C), lambda i: (i, 0)),
            edge_spec, edge_spec, edge_spec,
            edge_spec, edge_spec, edge_spec,
            pl.BlockSpec((K, 3), lambda i: (0, 0)),
            pl.BlockSpec((K * C, C), lambda i: (0, 0)),
        ],
        out_specs=pl.BlockSpec((B, C), lambda i: (i, 0)),
        out_shape=jax.ShapeDtypeStruct((nblk * B, C), jnp.float32),
        scratch_shapes=[pltpu.VMEM((B, K * C), jnp.float32)],
    )(featg, sxg, syg, szg, qxe, qye, qze, kp, wflat)


def kernel(query_points, support_points, neighbors_indices, features, wts,
           kernel_points):
    N, C = features.shape
    NN = neighbors_indices.shape[1]
    E = N * NN
    K = kernel_points.shape[0]
    idx_flat = neighbors_indices.reshape(-1).astype(jnp.int32)
    spx = support_points[:, 0]
    spy = support_points[:, 1]
    spz = support_points[:, 2]
    featg, sxg, syg, szg = _sc_gather(features, spx, spy, spz, idx_flat)
    return _chunk_tc(featg, sxg, syg, szg, query_points, kernel_points, wts,
                     NN, E)


def _chunk_tc(featg, sxg, syg, szg, qp_c, kernel_points, wts, NN, EC):
    PC, C = qp_c.shape[0], featg.shape[1]
    K = kernel_points.shape[0]
    EB = 1000 * NN
    nblk = EC // EB
    qxe = jnp.repeat(qp_c[:, 0], NN).reshape(nblk, 1, EB)
    qye = jnp.repeat(qp_c[:, 1], NN).reshape(nblk, 1, EB)
    qze = jnp.repeat(qp_c[:, 2], NN).reshape(nblk, 1, EB)
    wflat = wts.reshape(K * C, C)
    return _tc_compute(featg, sxg.reshape(nblk, 1, EB),
                       syg.reshape(nblk, 1, EB), szg.reshape(nblk, 1, EB),
                       qxe, qye, qze, kernel_points, wflat)
